# baseline (speedup 1.0000x reference)
import itertools

import jax
import jax.numpy as jnp
import numpy as np
from jax.experimental import pallas as pl
from jax.experimental.pallas import tpu as pltpu
from jax.experimental.pallas import tpu_sc as plsc

B = 64
N = 32
R = 16
F = 16
G = 4960

OCT = 8
NOCT = B // OCT


def _fused_body(x_ref, w_ref, o_ref, scr_ref, out_scr):
    y = jnp.dot(x_ref[0], w_ref[...], preferred_element_type=jnp.float32)
    scr_ref[0] = y[:, 0:128]
    scr_ref[1] = y[:, 128:256]
    scr_ref[2] = y[:, 256:384]
    off = 0
    for a in range(N - 2):
        for b2 in range(a + 1, N - 1):
            L = (N - 1) - b2
            r01 = scr_ref[0, a * N + b2, :]
            s02 = scr_ref[1, pl.ds(a * N + b2 + 1, L), :]
            s12 = scr_ref[2, pl.ds(b2 * N + b2 + 1, L), :]
            out_scr[pl.ds(off, L), :] = r01[None, :] + s02 + s12
            off += L
    for bi in range(OCT):
        o_ref[0, bi] = out_scr[:, pl.ds(bi * F, F)]


def _fused_tc(xab, w8):
    noct = xab.shape[0]
    return pl.pallas_call(
        _fused_body,
        grid=(noct,),
        in_specs=[
            pl.BlockSpec((1, N * N, 4 * R * OCT), lambda i: (i, 0, 0)),
            pl.BlockSpec((4 * R * OCT, 3 * OCT * F), lambda i: (0, 0)),
        ],
        out_specs=pl.BlockSpec((1, OCT, G, F), lambda i: (i, 0, 0, 0)),
        out_shape=jax.ShapeDtypeStruct((noct, OCT, G, F), jnp.float32),
        scratch_shapes=[
            pltpu.VMEM((3, N * N, OCT * F), jnp.float32),
            pltpu.VMEM((G, OCT * F), jnp.float32),
        ],
        compiler_params=pltpu.CompilerParams(
            dimension_semantics=("parallel",),
        ),
    )(xab, w8)


def kernel(inputs, filters):
    idx = jnp.arange(N)
    in_t = jnp.swapaxes(inputs, 1, 2)
    diag = inputs[:, idx, idx, :]
    d_v = jnp.broadcast_to(diag[:, None, :, :], (B, N, N, R))
    d_u = jnp.broadcast_to(diag[:, :, None, :], (B, N, N, R))
    comp = jnp.concatenate([inputs, in_t, d_v, d_u], axis=-1)
    xab = (
        comp.reshape(NOCT, OCT, N * N, 4 * R)
        .transpose(0, 2, 3, 1)
        .reshape(NOCT, N * N, 4 * R * OCT)
    )

    def fpq(p, q):
        return filters[:, p, q, :].T

    z = jnp.zeros((R, F), jnp.float32)
    wa01 = jnp.concatenate([fpq(0, 1), fpq(1, 0), fpq(1, 1), z], axis=0)
    wa02 = jnp.concatenate([fpq(0, 2), fpq(2, 0), z, fpq(0, 0)], axis=0)
    wa12 = jnp.concatenate([fpq(1, 2), fpq(2, 1), fpq(2, 2), z], axis=0)
    wa3 = jnp.stack([wa01, wa02, wa12], axis=1)
    w8 = jnp.einsum("kcf,bj->kbcjf", wa3, jnp.eye(OCT, dtype=jnp.float32))
    w8 = w8.reshape(4 * R * OCT, 3 * OCT * F)

    res = _fused_tc(xab, w8)
    return res.reshape(B, G, F)
